# baseline (speedup 1.0000x reference)
import functools

import jax
import jax.numpy as jnp
from jax import lax
from jax.experimental import pallas as pl
from jax.experimental.pallas import tpu as pltpu
from jax.experimental.pallas import tpu_sc as plsc

_CHUNK = 128
_LANES = 16


@functools.lru_cache(maxsize=None)
def _make_gather_kernel(B: int, D: int):
    info = plsc.get_sparse_core_info()
    nw = info.num_cores * info.num_subcores
    per_w = B // nw
    n_chunk = per_w // _CHUNK
    n_tr = D // 8
    n_tc = per_w // _CHUNK

    mesh = plsc.VectorSubcoreMesh(core_axis_name="c", subcore_axis_name="s")
    out = jax.ShapeDtypeStruct((n_tr, B // _CHUNK, 8, _CHUNK), jnp.float32)
    idx_t = pltpu.VMEM((n_chunk, _CHUNK), jnp.int32)
    rows_t = pltpu.VMEM((n_chunk, _CHUNK, D), jnp.float32)
    outT_t = pltpu.VMEM((n_tr, n_tc, 8, _CHUNK), jnp.float32)

    @functools.partial(
        pl.kernel,
        mesh=mesh,
        out_type=(out, out, out),
        scratch_types=[idx_t, idx_t, idx_t, rows_t, rows_t, rows_t,
                       outT_t, outT_t, outT_t, pltpu.SemaphoreType.DMA],
        compiler_params=pltpu.CompilerParams(
            use_tc_tiling_on_sc=False, needs_layout_passes=False),
    )
    def gather3(u_ix, p_ix, n_ix, uemb, iemb, out_u, out_p, out_n,
                idx_u, idx_p, idx_n, rows_u, rows_p, rows_n,
                outT_u, outT_p, outT_n, sem):
        wid = lax.axis_index("s") * info.num_cores + lax.axis_index("c")
        pltpu.sync_copy(u_ix.at[wid], idx_u)
        pltpu.sync_copy(p_ix.at[wid], idx_p)
        pltpu.sync_copy(n_ix.at[wid], idx_n)
        copies = []
        for idx, tab, rows in ((idx_u, uemb, rows_u),
                               (idx_p, iemb, rows_p),
                               (idx_n, iemb, rows_n)):
            for j in range(n_chunk):
                copies.append(pltpu.async_copy(tab.at[idx.at[j]], rows.at[j], sem))
        for cp in copies:
            cp.wait()

        lanes = lax.iota(jnp.int32, _LANES)
        for rows, outT in ((rows_u, outT_u), (rows_p, outT_p), (rows_n, outT_n)):
            def body(k, carry, rows=rows, outT=outT):
                kv = jnp.full((_LANES,), k, jnp.int32)
                for tr in range(n_tr):
                    for r in range(8):
                        cv = jnp.full((_LANES,), tr * 8 + r, jnp.int32)
                        for g in range(_CHUNK // _LANES):
                            vals = plsc.load_gather(
                                rows, [kv, lanes + g * _LANES, cv])
                            outT[tr, k, r, pl.ds(g * _LANES, _LANES)] = vals
                return carry
            lax.fori_loop(0, n_tc, body, 0)

        for outT, out_ref in ((outT_u, out_u), (outT_p, out_p), (outT_n, out_n)):
            for tr in range(n_tr):
                pltpu.sync_copy(outT.at[tr], out_ref.at[tr, pl.ds(wid * n_tc, n_tc)])

    return gather3, nw, n_chunk


def kernel(users, positives, negatives, User_Emb, Item_Emb):
    B = users.shape[0]
    D = User_Emb.shape[1]
    gather3, nw, n_chunk = _make_gather_kernel(B, D)
    shape3 = (nw, n_chunk, _CHUNK)
    u = users.astype(jnp.int32).reshape(shape3)
    p = positives.astype(jnp.int32).reshape(shape3)
    n = negatives.astype(jnp.int32).reshape(shape3)
    out_u, out_p, out_n = gather3(u, p, n, User_Emb, Item_Emb)

    def _untile(o4):
        return o4.transpose(1, 3, 0, 2).reshape(B, D)

    return (_untile(out_u), _untile(out_p), _untile(out_n))


# speedup vs baseline: 1.0103x; 1.0103x over previous
import functools

import jax
import jax.numpy as jnp
from jax import lax
from jax.experimental import pallas as pl
from jax.experimental.pallas import tpu as pltpu
from jax.experimental.pallas import tpu_sc as plsc

_CHUNK = 128


@functools.lru_cache(maxsize=None)
def _make_gather_kernel(B: int, D: int, n_users: int, n_items: int):
    info = plsc.get_sparse_core_info()
    nw = info.num_cores * info.num_subcores
    per_w = B // nw
    n_chunk = per_w // _CHUNK

    mesh = plsc.VectorSubcoreMesh(core_axis_name="c", subcore_axis_name="s")
    out = jax.ShapeDtypeStruct((nw, n_chunk, _CHUNK, D), jnp.float32)
    idx_t = pltpu.VMEM((n_chunk, _CHUNK), jnp.int32)
    rows_t = pltpu.VMEM((n_chunk, _CHUNK, D), jnp.float32)

    @functools.partial(
        pl.kernel,
        mesh=mesh,
        out_type=(out, out, out),
        scratch_types=[idx_t, idx_t, idx_t, rows_t, rows_t, rows_t,
                       pltpu.SemaphoreType.DMA],
        compiler_params=pltpu.CompilerParams(use_tc_tiling_on_sc=False),
    )
    def gather3(u_ix, p_ix, n_ix, uemb, iemb, out_u, out_p, out_n,
                idx_u, idx_p, idx_n, rows_u, rows_p, rows_n, sem):
        wid = lax.axis_index("s") * info.num_cores + lax.axis_index("c")
        pltpu.sync_copy(u_ix.at[wid], idx_u)
        pltpu.sync_copy(p_ix.at[wid], idx_p)
        pltpu.sync_copy(n_ix.at[wid], idx_n)
        copies = []
        for idx, tab, rows in ((idx_u, uemb, rows_u),
                               (idx_p, iemb, rows_p),
                               (idx_n, iemb, rows_n)):
            for j in range(n_chunk):
                copies.append(pltpu.async_copy(tab.at[idx.at[j]], rows.at[j], sem))
        for cp in copies:
            cp.wait()
        pltpu.sync_copy(rows_u, out_u.at[wid])
        pltpu.sync_copy(rows_p, out_p.at[wid])
        pltpu.sync_copy(rows_n, out_n.at[wid])

    return gather3, nw, n_chunk


def kernel(users, positives, negatives, User_Emb, Item_Emb):
    B = users.shape[0]
    D = User_Emb.shape[1]
    gather3, nw, n_chunk = _make_gather_kernel(
        B, D, User_Emb.shape[0], Item_Emb.shape[0])
    shape3 = (nw, n_chunk, _CHUNK)
    u = users.astype(jnp.int32).reshape(shape3)
    p = positives.astype(jnp.int32).reshape(shape3)
    n = negatives.astype(jnp.int32).reshape(shape3)
    out_u, out_p, out_n = gather3(u, p, n, User_Emb, Item_Emb)
    return (out_u.reshape(B, D), out_p.reshape(B, D), out_n.reshape(B, D))


# speedup vs baseline: 1.0148x; 1.0045x over previous
import functools

import jax
import jax.numpy as jnp
from jax import lax
from jax.experimental import pallas as pl
from jax.experimental.pallas import tpu as pltpu
from jax.experimental.pallas import tpu_sc as plsc

_CHUNK = 128


@functools.lru_cache(maxsize=None)
def _make_gather_kernel(B: int, D: int, n_streams: int):
    info = plsc.get_sparse_core_info()
    nw = info.num_cores * info.num_subcores
    per_w = B // nw
    n_chunk = per_w // _CHUNK

    mesh = plsc.VectorSubcoreMesh(core_axis_name="c", subcore_axis_name="s")
    out = jax.ShapeDtypeStruct((nw, n_chunk, _CHUNK, D), jnp.float32)
    idx_t = pltpu.VMEM((n_chunk, _CHUNK), jnp.int32)
    rows_t = pltpu.VMEM((n_chunk, _CHUNK, D), jnp.float32)

    @functools.partial(
        pl.kernel,
        mesh=mesh,
        out_type=(out,) * n_streams,
        scratch_types=[idx_t] * n_streams + [rows_t] * n_streams
        + [pltpu.SemaphoreType.DMA],
        compiler_params=pltpu.CompilerParams(use_tc_tiling_on_sc=False),
    )
    def gather(*args):
        ix_refs = args[:n_streams]
        tab = args[n_streams]
        out_refs = args[n_streams + 1:2 * n_streams + 1]
        idx_refs = args[2 * n_streams + 1:3 * n_streams + 1]
        rows_refs = args[3 * n_streams + 1:4 * n_streams + 1]
        sem = args[-1]
        wid = lax.axis_index("s") * info.num_cores + lax.axis_index("c")
        for ix, idx in zip(ix_refs, idx_refs):
            pltpu.sync_copy(ix.at[wid], idx)
        copies = []
        for idx, rows in zip(idx_refs, rows_refs):
            for j in range(n_chunk):
                copies.append(pltpu.async_copy(tab.at[idx.at[j]], rows.at[j], sem))
        for cp in copies:
            cp.wait()
        for rows, out_ref in zip(rows_refs, out_refs):
            pltpu.sync_copy(rows, out_ref.at[wid])

    return gather, nw, n_chunk


def kernel(users, positives, negatives, User_Emb, Item_Emb):
    B = users.shape[0]
    D = User_Emb.shape[1]
    gather1, nw, n_chunk = _make_gather_kernel(B, D, 1)
    gather2, _, _ = _make_gather_kernel(B, D, 2)
    shape3 = (nw, n_chunk, _CHUNK)
    u = users.astype(jnp.int32).reshape(shape3)
    p = positives.astype(jnp.int32).reshape(shape3)
    n = negatives.astype(jnp.int32).reshape(shape3)
    out_p, out_n = gather2(p, n, Item_Emb)
    (out_u,) = gather1(u, User_Emb)
    return (out_u.reshape(B, D), out_p.reshape(B, D), out_n.reshape(B, D))
